# baseline (device time: 62200 ns/iter reference)
import jax
import jax.numpy as jnp
from jax import lax
from jax.experimental import pallas as pl
from jax.experimental.pallas import tpu as pltpu

N_DEV = 32
ROW = 16
R_HOPS = ROW // 2
L_HOPS = ROW // 2 - 1


def kernel(q, k, v):
    s_per, d = q.shape
    scale = 1.0 / (d ** 0.5)

    def body(q_ref, k_ref, v_ref, out_ref,
             qmine, qother, rbuf, lbuf, sacc, racc,
             q_sems, br_send, br_recv, bl_send, bl_recv, c_sems):
        p = lax.axis_index("i")
        base = (p // ROW) * ROW
        w = p % ROW
        right = base + (w + 1) % ROW
        left = base + (w - 1) % ROW
        partner = (p + ROW) % N_DEV

        barrier_sem = pltpu.get_barrier_semaphore()
        for nbr in [left, right, partner]:
            pl.semaphore_signal(
                barrier_sem, inc=1,
                device_id=(nbr,), device_id_type=pl.DeviceIdType.MESH,
            )
        pl.semaphore_wait(barrier_sem, 3)

        k_bf = k_ref[:, :].astype(jnp.bfloat16)
        v_bf = v_ref[:, :].astype(jnp.bfloat16)
        rbuf[0, :s_per, :] = k_bf
        rbuf[0, s_per:, :] = v_bf
        lbuf[0, :s_per, :] = k_bf
        lbuf[0, s_per:, :] = v_bf
        qmine[:, :] = (q_ref[:, :] * scale).astype(jnp.bfloat16)

        def make(src, dst, send, recv, dev):
            return pltpu.make_async_remote_copy(
                src_ref=src, dst_ref=dst, send_sem=send, recv_sem=recv,
                device_id=(dev,), device_id_type=pl.DeviceIdType.MESH,
            )

        q_desc = make(qmine, qother, q_sems.at[0], q_sems.at[1], partner)
        r_desc = [
            make(rbuf.at[h], rbuf.at[h + 1], br_send.at[h + 1],
                 br_recv.at[h + 1], right)
            for h in range(R_HOPS)
        ]
        l_desc = [
            make(lbuf.at[h], lbuf.at[h + 1], bl_send.at[h + 1],
                 bl_recv.at[h + 1], left)
            for h in range(L_HOPS)
        ]
        c_desc = make(sacc, racc, c_sems.at[0], c_sems.at[1], partner)

        q_desc.start()
        r_desc[0].start()
        l_desc[0].start()

        ones = jnp.ones((s_per, d), dtype=jnp.bfloat16)
        acc_m = jnp.zeros((s_per, 2 * d), dtype=jnp.float32)
        acc_p = jnp.zeros((s_per, 2 * d), dtype=jnp.float32)

        def fold(acc, q_blk, buf, slot):
            k_h = buf[slot, :s_per, :]
            v_aug = jnp.concatenate([buf[slot, s_per:, :], ones], axis=1)
            scores = jax.lax.dot_general(
                q_blk, k_h,
                (((1,), (1,)), ((), ())),
                preferred_element_type=jnp.float32,
            )
            pr = jnp.exp(scores.astype(jnp.bfloat16))
            pv = jax.lax.dot_general(
                pr, v_aug,
                (((1,), (0,)), ((), ())),
                preferred_element_type=jnp.float32,
            )
            return acc + pv

        qm = qmine[:, :]
        acc_m = fold(acc_m, qm, rbuf, 0)
        q_desc.wait_recv()
        qo = qother[:, :]
        acc_p = fold(acc_p, qo, rbuf, 0)

        for h in range(1, R_HOPS + 1):
            r_desc[h - 1].wait_recv()
            if h < R_HOPS:
                r_desc[h].start()
                acc_m = fold(acc_m, qm, rbuf, h)
                acc_p = fold(acc_p, qo, rbuf, h)
            else:
                acc_p = fold(acc_p, qo, rbuf, h)
                sacc[:, :] = acc_p.astype(jnp.bfloat16)
                c_desc.start()
                acc_m = fold(acc_m, qm, rbuf, h)
            if h <= L_HOPS:
                l_desc[h - 1].wait_recv()
                if h < L_HOPS:
                    l_desc[h].start()
                acc_m = fold(acc_m, qm, lbuf, h)
                acc_p = fold(acc_p, qo, lbuf, h)

        c_desc.wait_recv()
        aug0 = acc_m + racc[:, :].astype(jnp.float32)
        out_ref[:, :] = aug0[:, :d] / aug0[:, d:d + 1]

        for desc in [q_desc, c_desc] + r_desc + l_desc:
            desc.wait_send()

    return pl.pallas_call(
        body,
        out_shape=jax.ShapeDtypeStruct((s_per, d), jnp.float32),
        in_specs=[
            pl.BlockSpec(memory_space=pltpu.VMEM),
            pl.BlockSpec(memory_space=pltpu.VMEM),
            pl.BlockSpec(memory_space=pltpu.VMEM),
        ],
        out_specs=pl.BlockSpec(memory_space=pltpu.VMEM),
        scratch_shapes=[
            pltpu.VMEM((s_per, d), jnp.bfloat16),
            pltpu.VMEM((s_per, d), jnp.bfloat16),
            pltpu.VMEM((R_HOPS + 1, 2 * s_per, d), jnp.bfloat16),
            pltpu.VMEM((L_HOPS + 1, 2 * s_per, d), jnp.bfloat16),
            pltpu.VMEM((s_per, 2 * d), jnp.bfloat16),
            pltpu.VMEM((s_per, 2 * d), jnp.bfloat16),
            pltpu.SemaphoreType.DMA((2,)),
            pltpu.SemaphoreType.DMA((R_HOPS + 1,)),
            pltpu.SemaphoreType.DMA((R_HOPS + 1,)),
            pltpu.SemaphoreType.DMA((L_HOPS + 1,)),
            pltpu.SemaphoreType.DMA((L_HOPS + 1,)),
            pltpu.SemaphoreType.DMA((2,)),
        ],
        compiler_params=pltpu.CompilerParams(collective_id=0),
    )(q, k, v)
